# Initial kernel scaffold; baseline (speedup 1.0000x reference)
import jax
import jax.numpy as jnp
from jax.experimental import pallas as pl


def kernel(x0, x1, x2, select1, select2, edge_index, edge_weight, gcn_W, gcn_Wr, gcn_b, W1_edge, W1_mid, Wskip_edge, Wskip_mid, W2, b1, b2):
    raise NotImplementedError("write your pallas kernel here")


# speedup vs baseline: 12.3212x; 12.3212x over previous
import functools

import jax
import jax.numpy as jnp
from jax import lax
from jax.experimental import pallas as pl
from jax.experimental.pallas import tpu as pltpu
from jax.experimental.pallas import tpu_sc as plsc

N0, N1, N2 = 10000, 1000, 100
D = 128
E = 320000
L = 2

NC, NS = 2, 16
NW = NC * NS
N0P = 10240
RPT = N0P // NS
EP = NW * 10240
EPW = EP // NW
K = 256
KJ = K // 128
NCH = EPW // K

_MESH = dict(core_axis_name="c", subcore_axis_name="s")
_F32 = jnp.float32


def _deg_body(src_hbm, dst_hbm, w_hbm, out_hbm, deg_o, deg_i, sidx, didx,
              wv, zbuf):
    c = lax.axis_index("c")
    s = lax.axis_index("s")
    wid = c * NS + s

    def _zero(i, carry):
        zbuf[pl.ds(i * 16, 16)] = jnp.zeros((16,), _F32)
        return carry

    lax.fori_loop(0, RPT // 16, _zero, 0)
    pltpu.sync_copy(zbuf, deg_o.at[pl.ds(s * RPT, RPT)])
    pltpu.sync_copy(zbuf, deg_i.at[pl.ds(s * RPT, RPT)])
    plsc.subcore_barrier()

    r0 = wid * (EPW // 128)
    pltpu.sync_copy(src_hbm.at[pl.ds(r0, EPW // 128)], sidx)
    pltpu.sync_copy(dst_hbm.at[pl.ds(r0, EPW // 128)], didx)
    pltpu.sync_copy(w_hbm.at[pl.ds(r0, EPW // 128)], wv)
    for j in range(EPW // 128):
        pltpu.sync_copy(wv.at[j], deg_o.at[sidx.at[j]], add=True)
        pltpu.sync_copy(wv.at[j], deg_i.at[didx.at[j]], add=True)
    plsc.subcore_barrier()

    @pl.when(s == 0)
    def _():
        pltpu.sync_copy(deg_o, out_hbm.at[c, 0])
        pltpu.sync_copy(deg_i, out_hbm.at[c, 1])


_deg_call = pl.kernel(
    _deg_body,
    out_type=jax.ShapeDtypeStruct((NC, 2, N0P), _F32),
    mesh=plsc.VectorSubcoreMesh(**_MESH),
    scratch_types=[
        pltpu.VMEM_SHARED((N0P,), _F32),
        pltpu.VMEM_SHARED((N0P,), _F32),
        pltpu.VMEM((EPW // 128, 128), jnp.int32),
        pltpu.VMEM((EPW // 128, 128), jnp.int32),
        pltpu.VMEM((EPW // 128, 128), _F32),
        pltpu.VMEM((RPT,), _F32),
    ],
)


def _agg_body(xw_hbm, src_hbm, dst_hbm, w_hbm, out_hbm, acc, sidx, didx,
              wv, rows, gsem):
    c = lax.axis_index("c")
    s = lax.axis_index("s")
    wid = c * NS + s

    def _zero(i, carry):
        for f in range(8):
            rows[i, pl.ds(f * 16, 16)] = jnp.zeros((16,), _F32)
        return carry

    lax.fori_loop(0, K, _zero, 0)
    off = 0
    while off < RPT:
        n = min(K, RPT - off)
        pltpu.sync_copy(rows.at[pl.ds(0, n)],
                        acc.at[pl.ds(s * RPT + off, n)])
        off += n
    plsc.subcore_barrier()

    erow0 = wid * (EPW // 128)

    def _chunk(ci, carry):
        r0 = erow0 + ci * KJ
        pltpu.sync_copy(src_hbm.at[pl.ds(r0, KJ)], sidx)
        pltpu.sync_copy(dst_hbm.at[pl.ds(r0, KJ)], didx)
        pltpu.sync_copy(w_hbm.at[pl.ds(wid * EPW + ci * K, K)], wv)
        descs = [
            pltpu.async_copy(xw_hbm.at[sidx.at[j]],
                             rows.at[pl.ds(j * 128, 128)], gsem)
            for j in range(KJ)
        ]
        for d in descs:
            d.wait()

        def _scale(r16, carry2):
            w16 = wv[pl.ds(r16 * 16, 16)]
            for t in range(16):
                r = r16 * 16 + t
                wt = jnp.full((16,), w16[t], _F32)
                for f in range(8):
                    rows[r, pl.ds(f * 16, 16)] = (
                        rows[r, pl.ds(f * 16, 16)] * wt)
            return carry2

        lax.fori_loop(0, K // 16, _scale, 0)
        for j in range(KJ):
            pltpu.sync_copy(rows.at[pl.ds(j * 128, 128)],
                            acc.at[didx.at[j]], add=True)
        return carry

    lax.fori_loop(0, NCH, _chunk, 0)
    plsc.subcore_barrier()
    pltpu.sync_copy(acc.at[pl.ds(s * RPT, RPT)],
                    out_hbm.at[c, pl.ds(s * RPT, RPT)])


_agg_call = pl.kernel(
    _agg_body,
    out_type=jax.ShapeDtypeStruct((NC, N0P, D), _F32),
    mesh=plsc.VectorSubcoreMesh(**_MESH),
    scratch_types=[
        pltpu.VMEM_SHARED((N0P, D), _F32),
        pltpu.VMEM((KJ, 128), jnp.int32),
        pltpu.VMEM((KJ, 128), jnp.int32),
        pltpu.VMEM((K,), _F32),
        pltpu.VMEM((K, D), _F32),
        pltpu.SemaphoreType.DMA,
    ],
)


def _dot(a, b):
    return jnp.dot(a, b, preferred_element_type=_F32)


def _dot_t(a, b):
    return lax.dot_general(a, b, (((0,), (0,)), ((), ())),
                           preferred_element_type=_F32)


def _silu(x):
    return x * jax.nn.sigmoid(x)


def _degfin_body(p_ref, dd_ref):
    deg_o = p_ref[0, 0] + p_ref[1, 0]
    deg_i = p_ref[0, 1] + p_ref[1, 1]
    dd_ref[0] = jnp.where(deg_o > 0, lax.rsqrt(jnp.maximum(deg_o, 1e-12)),
                          0.0)
    dd_ref[1] = jnp.where(deg_i > 0, lax.rsqrt(jnp.maximum(deg_i, 1e-12)),
                          0.0)


_degfin_call = pl.pallas_call(
    _degfin_body,
    out_shape=jax.ShapeDtypeStruct((2, N0P // 128, 128), _F32),
)

B0 = 1000


def _pre_body(o0_ref, w_ref, wr_ref, b_ref, dos_ref, xw_ref, xr_ref):
    o0 = o0_ref[...]
    xw_ref[...] = _dot(o0, w_ref[...]) * dos_ref[...]
    xr_ref[...] = _dot(o0, wr_ref[...]) + b_ref[...]


_pre_call = pl.pallas_call(
    _pre_body,
    grid=(N0 // B0,),
    in_specs=[
        pl.BlockSpec((B0, D), lambda i: (i, 0)),
        pl.BlockSpec((D, D), lambda i: (0, 0)),
        pl.BlockSpec((D, D), lambda i: (0, 0)),
        pl.BlockSpec((1, D), lambda i: (0, 0)),
        pl.BlockSpec((B0, 1), lambda i: (i, 0)),
    ],
    out_specs=[
        pl.BlockSpec((B0, D), lambda i: (i, 0)),
        pl.BlockSpec((B0, D), lambda i: (i, 0)),
    ],
    out_shape=[
        jax.ShapeDtypeStruct((N0, D), _F32),
        jax.ShapeDtypeStruct((N0, D), _F32),
    ],
)


def _cmb_body(p_ref, xr_ref, dis_ref, o_ref):
    z = (p_ref[0] + p_ref[1]) * dis_ref[...] + xr_ref[...]
    o_ref[...] = _silu(z)


_cmb_call = pl.pallas_call(
    _cmb_body,
    grid=(N0 // B0,),
    in_specs=[
        pl.BlockSpec((2, B0, D), lambda i: (0, i, 0)),
        pl.BlockSpec((B0, D), lambda i: (i, 0)),
        pl.BlockSpec((B0, 1), lambda i: (i, 0)),
    ],
    out_specs=pl.BlockSpec((B0, D), lambda i: (i, 0)),
    out_shape=jax.ShapeDtypeStruct((N0, D), _F32),
)


def _sel2_body(s2_ref, o1_ref, o2_ref, s2o2_ref, s2to1_ref):
    s2 = s2_ref[...]
    s2o2_ref[...] = _dot(s2, o2_ref[...])
    s2to1_ref[...] = _dot_t(s2, o1_ref[...])


_sel2_call = pl.pallas_call(
    _sel2_body,
    out_shape=[
        jax.ShapeDtypeStruct((N1, D), _F32),
        jax.ShapeDtypeStruct((N2, D), _F32),
    ],
)


def _sel1_body(s1_ref, o1_ref, o0_ref, s1o1_ref, s1to0_ref):
    i = pl.program_id(0)
    s1 = s1_ref[...]
    s1o1_ref[...] = _dot(s1, o1_ref[...])
    part = _dot_t(s1, o0_ref[...])

    @pl.when(i == 0)
    def _():
        s1to0_ref[...] = part

    @pl.when(i > 0)
    def _():
        s1to0_ref[...] += part


_sel1_call = pl.pallas_call(
    _sel1_body,
    grid=(N0 // B0,),
    in_specs=[
        pl.BlockSpec((B0, N1), lambda i: (i, 0)),
        pl.BlockSpec((N1, D), lambda i: (0, 0)),
        pl.BlockSpec((B0, D), lambda i: (i, 0)),
    ],
    out_specs=[
        pl.BlockSpec((B0, D), lambda i: (i, 0)),
        pl.BlockSpec((N1, D), lambda i: (0, 0)),
    ],
    out_shape=[
        jax.ShapeDtypeStruct((N0, D), _F32),
        jax.ShapeDtypeStruct((N1, D), _F32),
    ],
)


def _mlp0_body(o0_ref, s1o1_ref, w1a_ref, w1b_ref, wsa_ref, wsb_ref,
               w2_ref, b1_ref, b2_ref, out_ref):
    o0 = o0_ref[...]
    s1o1 = s1o1_ref[...]
    h = _silu(_dot(o0, w1a_ref[...]) + _dot(s1o1, w1b_ref[...]) +
              b1_ref[...])
    out_ref[...] = (_dot(h, w2_ref[...]) + b2_ref[...] +
                    _dot(o0, wsa_ref[...]) + _dot(s1o1, wsb_ref[...]))


_mlp0_call = pl.pallas_call(
    _mlp0_body,
    grid=(N0 // B0,),
    in_specs=[
        pl.BlockSpec((B0, D), lambda i: (i, 0)),
        pl.BlockSpec((B0, D), lambda i: (i, 0)),
        pl.BlockSpec((D, D), lambda i: (0, 0)),
        pl.BlockSpec((D, D), lambda i: (0, 0)),
        pl.BlockSpec((D, D), lambda i: (0, 0)),
        pl.BlockSpec((D, D), lambda i: (0, 0)),
        pl.BlockSpec((D, D), lambda i: (0, 0)),
        pl.BlockSpec((1, D), lambda i: (0, 0)),
        pl.BlockSpec((1, D), lambda i: (0, 0)),
    ],
    out_specs=pl.BlockSpec((B0, D), lambda i: (i, 0)),
    out_shape=jax.ShapeDtypeStruct((N0, D), _F32),
)


def _mlp12_body(o1_ref, s1to0_ref, s2o2_ref, o2_ref, s2to1_ref,
                w1ma_ref, w1mb_ref, w1mc_ref, wsma_ref, wsmb_ref, wsmc_ref,
                w2m_ref, b1m_ref, b2m_ref,
                w1ea_ref, w1eb_ref, wsea_ref, wseb_ref, w2e_ref, b1e_ref,
                b2e_ref, o1_out, o2_out):
    o1 = o1_ref[...]
    s1to0 = s1to0_ref[...]
    s2o2 = s2o2_ref[...]
    h1 = _silu(_dot(o1, w1ma_ref[...]) + _dot(s1to0, w1mb_ref[...]) +
               _dot(s2o2, w1mc_ref[...]) + b1m_ref[...])
    o1_out[...] = (_dot(h1, w2m_ref[...]) + b2m_ref[...] +
                   _dot(o1, wsma_ref[...]) + _dot(s1to0, wsmb_ref[...]) +
                   _dot(s2o2, wsmc_ref[...]))
    o2 = o2_ref[...]
    s2to1 = s2to1_ref[...]
    h2 = _silu(_dot(o2, w1ea_ref[...]) + _dot(s2to1, w1eb_ref[...]) +
               b1e_ref[...])
    o2_out[...] = (_dot(h2, w2e_ref[...]) + b2e_ref[...] +
                   _dot(o2, wsea_ref[...]) + _dot(s2to1, wseb_ref[...]))


_mlp12_call = pl.pallas_call(
    _mlp12_body,
    out_shape=[
        jax.ShapeDtypeStruct((N1, D), _F32),
        jax.ShapeDtypeStruct((N2, D), _F32),
    ],
)


def kernel(x0, x1, x2, select1, select2, edge_index, edge_weight, gcn_W,
           gcn_Wr, gcn_b, W1_edge, W1_mid, Wskip_edge, Wskip_mid, W2, b1,
           b2):
    pad = EP - E
    fill = (jnp.arange(pad, dtype=jnp.int32) * 131) % N0
    src = jnp.concatenate([edge_index[0], fill])
    dst = jnp.concatenate([edge_index[1], fill])
    w = jnp.concatenate([edge_weight, jnp.zeros((pad,), _F32)])
    src2d = src.reshape(EP // 128, 128)
    dst2d = dst.reshape(EP // 128, 128)
    w2d = w.reshape(EP // 128, 128)

    degp = _deg_call(src2d, dst2d, w2d)
    dd = _degfin_call(degp.reshape(2, 2, N0P // 128, 128))
    dd = dd.reshape(2, N0P)
    dos = dd[0, :N0, None]
    dis = dd[1, :N0, None]

    o0, o1, o2 = x0, x1, x2
    for l in range(L):
        xw, xr = _pre_call(o0, gcn_W[l], gcn_Wr[l], gcn_b[l][None], dos)
        part = _agg_call(xw, src2d, dst2d, w)
        o0g = _cmb_call(part, xr, dis)
        s2o2, s2to1 = _sel2_call(select2, o1, o2)
        s1o1, s1to0 = _sel1_call(select1, o1, o0g)
        o0 = _mlp0_call(o0g, s1o1, W1_edge[l, 0, :D], W1_edge[l, 0, D:],
                        Wskip_edge[l, 0, :D], Wskip_edge[l, 0, D:],
                        W2[l, 0], b1[l, 0][None], b2[l, 0][None])
        o1, o2 = _mlp12_call(
            o1, s1to0, s2o2, o2, s2to1,
            W1_mid[l, :D], W1_mid[l, D:2 * D], W1_mid[l, 2 * D:],
            Wskip_mid[l, :D], Wskip_mid[l, D:2 * D], Wskip_mid[l, 2 * D:],
            W2[l, 1], b1[l, 1][None], b2[l, 1][None],
            W1_edge[l, 1, :D], W1_edge[l, 1, D:],
            Wskip_edge[l, 1, :D], Wskip_edge[l, 1, D:],
            W2[l, 2], b1[l, 2][None], b2[l, 2][None])
    return (o0, o1, o2)
